# Initial kernel scaffold; baseline (speedup 1.0000x reference)
import jax
import jax.numpy as jnp
from jax.experimental import pallas as pl


def kernel(inputs, edge_index, W1, b1, W2, b2):
    raise NotImplementedError("write your pallas kernel here")


# speedup vs baseline: 9.9132x; 9.9132x over previous
import functools

import jax
import jax.numpy as jnp
from jax import lax
from jax.experimental import pallas as pl
from jax.experimental.pallas import tpu as pltpu
from jax.experimental.pallas import tpu_sc as plsc

N = 10000
NP = 10240
E = 320000
FIN = 128
FHID = 128
FOUT = 64
NC = 2
NS = 16
NW = NC * NS
EPT = E // NW
K = 80
NCHUNK = EPT // K
ROWS_PT = NP // NS
DW = 8
RB = NP // 8
_MESH = plsc.VectorSubcoreMesh(core_axis_name="c", subcore_axis_name="s")


@functools.partial(
    pl.kernel,
    out_type=[
        jax.ShapeDtypeStruct((NC, NP, DW), jnp.float32),
        jax.ShapeDtypeStruct((NC, NP, DW), jnp.float32),
    ],
    mesh=_MESH,
    compiler_params=pltpu.CompilerParams(use_tc_tiling_on_sc=False),
    scratch_types=[
        pltpu.VMEM((NCHUNK, K), jnp.int32),
        pltpu.VMEM((NCHUNK, K), jnp.int32),
        pltpu.VMEM((K, DW), jnp.float32),
        pltpu.VMEM_SHARED((NP, DW), jnp.float32),
        pltpu.VMEM_SHARED((NP, DW), jnp.float32),
    ],
)
def _degrees(src_hbm, dst_hbm, ones_hbm, zeros_hbm, dego_hbm, degi_hbm,
             src_v, dst_v, ones_v, dego_s, degi_s):
    c = lax.axis_index("c")
    s = lax.axis_index("s")
    wid = c * NS + s
    lo = s * ROWS_PT
    pltpu.sync_copy(ones_hbm, ones_v)
    pltpu.sync_copy(zeros_hbm, dego_s.at[pl.ds(lo, ROWS_PT)])
    pltpu.sync_copy(zeros_hbm, degi_s.at[pl.ds(lo, ROWS_PT)])
    pltpu.sync_copy(src_hbm.at[wid], src_v)
    pltpu.sync_copy(dst_hbm.at[wid], dst_v)
    plsc.subcore_barrier()

    @pl.loop(0, NCHUNK)
    def _(j):
        pltpu.sync_copy(ones_v, dego_s.at[src_v.at[j]], add=True)
        pltpu.sync_copy(ones_v, degi_s.at[dst_v.at[j]], add=True)

    plsc.subcore_barrier()
    pltpu.sync_copy(dego_s.at[pl.ds(lo, ROWS_PT)], dego_hbm.at[c, pl.ds(lo, ROWS_PT)])
    pltpu.sync_copy(degi_s.at[pl.ds(lo, ROWS_PT)], degi_hbm.at[c, pl.ds(lo, ROWS_PT)])


def _make_propagate(F):
    zrows = 32

    @functools.partial(
        pl.kernel,
        out_type=jax.ShapeDtypeStruct((NC, NP, F), jnp.float32),
        mesh=_MESH,
        compiler_params=pltpu.CompilerParams(use_tc_tiling_on_sc=False),
        scratch_types=[
            pltpu.VMEM((NCHUNK, K), jnp.int32),
            pltpu.VMEM((NCHUNK, K), jnp.int32),
            pltpu.VMEM((K, F), jnp.float32),
            pltpu.VMEM((K, F), jnp.float32),
            pltpu.VMEM((zrows, F), jnp.float32),
            pltpu.VMEM_SHARED((NP, F), jnp.float32),
            pltpu.SemaphoreType.DMA,
            pltpu.SemaphoreType.DMA,
        ],
    )
    def _propagate(h_hbm, src_hbm, dst_hbm, out_hbm,
                   src_v, dst_v, rows_a, rows_b, zbuf, agg, sem_a, sem_b):
        c = lax.axis_index("c")
        s = lax.axis_index("s")
        wid = c * NS + s
        zero16 = jnp.zeros((16,), jnp.float32)

        @pl.loop(0, zrows)
        def _(i):
            for j in range(F // 16):
                zbuf[i, pl.ds(j * 16, 16)] = zero16

        @pl.loop(0, ROWS_PT // zrows)
        def _(i):
            pltpu.sync_copy(zbuf, agg.at[pl.ds(s * ROWS_PT + i * zrows, zrows)])

        pltpu.sync_copy(src_hbm.at[wid], src_v)
        pltpu.sync_copy(dst_hbm.at[wid], dst_v)
        plsc.subcore_barrier()

        pltpu.async_copy(h_hbm.at[src_v.at[0]], rows_a, sem_a)

        @pl.loop(0, (NCHUNK - 1) // 2)
        def _(p):
            j = 2 * p
            pltpu.make_async_copy(h_hbm.at[src_v.at[j]], rows_a, sem_a).wait()
            pltpu.async_copy(h_hbm.at[src_v.at[j + 1]], rows_b, sem_b)
            pltpu.sync_copy(rows_a, agg.at[dst_v.at[j]], add=True)
            pltpu.make_async_copy(h_hbm.at[src_v.at[j + 1]], rows_b, sem_b).wait()
            pltpu.async_copy(h_hbm.at[src_v.at[j + 2]], rows_a, sem_a)
            pltpu.sync_copy(rows_b, agg.at[dst_v.at[j + 1]], add=True)

        pltpu.make_async_copy(h_hbm.at[src_v.at[NCHUNK - 1]], rows_a, sem_a).wait()
        pltpu.sync_copy(rows_a, agg.at[dst_v.at[NCHUNK - 1]], add=True)

        plsc.subcore_barrier()
        lo = s * ROWS_PT
        pltpu.sync_copy(agg.at[pl.ds(lo, ROWS_PT)], out_hbm.at[c, pl.ds(lo, ROWS_PT)])

    return _propagate


_prop_hid = _make_propagate(FHID)
_prop_out = _make_propagate(FOUT)


def _norms_body(x_ref, dgo_ref, dgi_ref, h0_ref, no_ref, ni_ref):
    dgo = (dgo_ref[0] + dgo_ref[1])[:, 0:1]
    dgi = (dgi_ref[0] + dgi_ref[1])[:, 0:1]
    no = jnp.where(dgo > 0, lax.rsqrt(jnp.maximum(dgo, 1.0)), 0.0)
    ni = jnp.where(dgi > 0, lax.rsqrt(jnp.maximum(dgi, 1.0)), 0.0)
    no_ref[...] = no
    ni_ref[...] = ni
    h0_ref[...] = x_ref[...] * no


_norms_call = pl.pallas_call(
    _norms_body,
    grid=(NP // RB,),
    in_specs=[
        pl.BlockSpec((RB, FIN), lambda i: (i, 0)),
        pl.BlockSpec((NC, RB, DW), lambda i: (0, i, 0)),
        pl.BlockSpec((NC, RB, DW), lambda i: (0, i, 0)),
    ],
    out_specs=[
        pl.BlockSpec((RB, FIN), lambda i: (i, 0)),
        pl.BlockSpec((RB, 1), lambda i: (i, 0)),
        pl.BlockSpec((RB, 1), lambda i: (i, 0)),
    ],
    out_shape=[
        jax.ShapeDtypeStruct((NP, FIN), jnp.float32),
        jax.ShapeDtypeStruct((NP, 1), jnp.float32),
        jax.ShapeDtypeStruct((NP, 1), jnp.float32),
    ],
)


def _dense_body(p_ref, ni_ref, no_ref, w1_ref, b1_ref, w2_ref, t_ref):
    agg = (p_ref[0] + p_ref[1]) * ni_ref[...]
    h1 = jnp.dot(agg, w1_ref[...], preferred_element_type=jnp.float32)
    h1 = jnp.maximum(h1 + b1_ref[...], 0.0)
    t = jnp.dot(h1, w2_ref[...], preferred_element_type=jnp.float32)
    t_ref[...] = t * no_ref[...]


_dense_call = pl.pallas_call(
    _dense_body,
    grid=(NP // RB,),
    in_specs=[
        pl.BlockSpec((NC, RB, FHID), lambda i: (0, i, 0)),
        pl.BlockSpec((RB, 1), lambda i: (i, 0)),
        pl.BlockSpec((RB, 1), lambda i: (i, 0)),
        pl.BlockSpec((FIN, FHID), lambda i: (0, 0)),
        pl.BlockSpec((1, FHID), lambda i: (0, 0)),
        pl.BlockSpec((FHID, FOUT), lambda i: (0, 0)),
    ],
    out_specs=pl.BlockSpec((RB, FOUT), lambda i: (i, 0)),
    out_shape=jax.ShapeDtypeStruct((NP, FOUT), jnp.float32),
)


def _final_body(q_ref, ni_ref, b2_ref, out_ref):
    out_ref[...] = (q_ref[0] + q_ref[1]) * ni_ref[...] + b2_ref[...]


_final_call = pl.pallas_call(
    _final_body,
    grid=(NP // RB,),
    in_specs=[
        pl.BlockSpec((NC, RB, FOUT), lambda i: (0, i, 0)),
        pl.BlockSpec((RB, 1), lambda i: (i, 0)),
        pl.BlockSpec((1, FOUT), lambda i: (0, 0)),
    ],
    out_specs=pl.BlockSpec((RB, FOUT), lambda i: (i, 0)),
    out_shape=jax.ShapeDtypeStruct((NP, FOUT), jnp.float32),
)


def kernel(inputs, edge_index, W1, b1, W2, b2):
    src = edge_index[0].astype(jnp.int32).reshape(NW, NCHUNK, K)
    dst = edge_index[1].astype(jnp.int32).reshape(NW, NCHUNK, K)
    x_pad = jnp.pad(inputs, ((0, NP - N), (0, 0)))
    ones_kw = jnp.ones((K, DW), jnp.float32)
    zeros_rw = jnp.zeros((ROWS_PT, DW), jnp.float32)
    dego, degi = _degrees(src, dst, ones_kw, zeros_rw)
    h0, no, ni = _norms_call(x_pad, dego, degi)
    p = _prop_hid(h0, src, dst)
    t = _dense_call(p, ni, no, W1, b1.reshape(1, FHID), W2)
    q = _prop_out(t, src, dst)
    out = _final_call(q, ni, b2.reshape(1, FOUT))
    return out[:N]
